# Initial kernel scaffold; baseline (speedup 1.0000x reference)
import jax
import jax.numpy as jnp
from jax.experimental import pallas as pl


def kernel(x, edge_index, W, b, Wd):
    raise NotImplementedError("write your pallas kernel here")


# speedup vs baseline: 4.2187x; 4.2187x over previous
import functools

import jax
import jax.numpy as jnp
from jax import lax
from jax.experimental import pallas as pl
from jax.experimental.pallas import tpu as pltpu
from jax.experimental.pallas import tpu_sc as plsc

N = 10000
E = 320000
D = 128
NC = 2
NS = 16
CH = 80
NP = 10240
EPT = E // NS
NCHUNK = EPT // CH
RPT = NP // NS
ZPT = NP // (NC * NS)


def _sc_prep_body(edge_hbm, x_hbm, perm_hbm, zeros128_hbm, ones128_hbm,
                  deg_hbm, z_hbm,
                  deg_acc, idx_v, buf, rows_v, sem):
    c = lax.axis_index("c")
    s = lax.axis_index("s")
    wid = s * NC + c
    pltpu.sync_copy(zeros128_hbm, buf)
    for k in range(RPT // CH):
        pltpu.sync_copy(buf, deg_acc.at[pl.ds(s * RPT + k * CH, CH)])
    pltpu.sync_copy(ones128_hbm, buf)
    plsc.subcore_barrier()

    def deg_step(i, carry):
        base = c * E + s * EPT + i * CH
        pltpu.sync_copy(edge_hbm.at[pl.ds(base, CH)], idx_v)
        pltpu.sync_copy(buf, deg_acc.at[idx_v], add=True)
        return carry

    lax.fori_loop(0, NCHUNK, deg_step, 0)

    def z_step(k, carry):
        zb = wid * ZPT + k * CH
        pltpu.sync_copy(perm_hbm.at[pl.ds(zb, CH)], idx_v)
        pltpu.async_copy(x_hbm.at[idx_v], rows_v, sem).wait()
        pltpu.sync_copy(rows_v, z_hbm.at[pl.ds(zb, CH)])
        return carry

    lax.fori_loop(0, ZPT // CH, z_step, 0)
    plsc.subcore_barrier()

    for k in range(RPT // CH):
        rb = s * RPT + k * CH
        pltpu.sync_copy(deg_acc.at[pl.ds(rb, CH)], buf)
        pltpu.sync_copy(buf, deg_hbm.at[c, pl.ds(rb, CH)])


_sc_prep = pl.kernel(
    _sc_prep_body,
    out_type=[jax.ShapeDtypeStruct((NC, NP, D), jnp.float32),
              jax.ShapeDtypeStruct((NP, D), jnp.float32)],
    mesh=plsc.VectorSubcoreMesh(core_axis_name="c", subcore_axis_name="s"),
    scratch_types=[
        pltpu.VMEM_SHARED((NP, D), jnp.float32),
        pltpu.VMEM((CH,), jnp.int32),
        pltpu.VMEM((CH, D), jnp.float32),
        pltpu.VMEM((CH, D), jnp.float32),
        pltpu.SemaphoreType.DMA,
    ],
)


def _sc_prop_body(t_hbm, srcs_hbm, dst_hbm, zeros128_hbm,
                  agg_hbm,
                  acc, src_v, dst_v, rows_v, sem):
    c = lax.axis_index("c")
    s = lax.axis_index("s")
    pltpu.sync_copy(zeros128_hbm, rows_v)
    for k in range(RPT // CH):
        pltpu.sync_copy(rows_v, acc.at[pl.ds(s * RPT + k * CH, CH)])
    plsc.subcore_barrier()

    def step(i, carry):
        base = s * EPT + i * CH
        pltpu.sync_copy(srcs_hbm.at[pl.ds(c * E + base, CH)], src_v)
        pltpu.sync_copy(dst_hbm.at[pl.ds(base, CH)], dst_v)
        pltpu.async_copy(t_hbm.at[src_v], rows_v, sem).wait()
        pltpu.sync_copy(rows_v, acc.at[dst_v], add=True)
        return carry

    lax.fori_loop(0, NCHUNK, step, 0)
    plsc.subcore_barrier()

    for k in range(RPT // CH):
        rb = s * RPT + k * CH
        pltpu.sync_copy(acc.at[pl.ds(rb, CH)], rows_v)
        pltpu.sync_copy(rows_v, agg_hbm.at[c, pl.ds(rb, CH)])


_sc_prop = pl.kernel(
    _sc_prop_body,
    out_type=jax.ShapeDtypeStruct((NC, NP, D), jnp.float32),
    mesh=plsc.VectorSubcoreMesh(core_axis_name="c", subcore_axis_name="s"),
    scratch_types=[
        pltpu.VMEM_SHARED((NP, D), jnp.float32),
        pltpu.VMEM((CH,), jnp.int32),
        pltpu.VMEM((CH,), jnp.int32),
        pltpu.VMEM((CH, D), jnp.float32),
        pltpu.SemaphoreType.DMA,
    ],
)


_BS = 1024


def _tc_prep_body(x_ref, z_ref, deg_ref, t_ref):
    u = lax.rsqrt(jnp.maximum(deg_ref[:, 0:1], 1.0))
    t_ref[0] = x_ref[...] * u
    t_ref[1] = z_ref[...] * u


def _tc_prep(x_pad, z, deg_out):
    return pl.pallas_call(
        _tc_prep_body,
        grid=(NP // _BS,),
        in_specs=[
            pl.BlockSpec((_BS, D), lambda g: (g, 0)),
            pl.BlockSpec((_BS, D), lambda g: (g, 0)),
            pl.BlockSpec((_BS, D), lambda g: (g, 0)),
        ],
        out_specs=pl.BlockSpec((2, _BS, D), lambda g: (0, g, 0)),
        out_shape=jax.ShapeDtypeStruct((2, NP, D), jnp.float32),
    )(x_pad, z, deg_out)


def _tc_loss_body(agg_ref, degin_ref, w_ref, b_ref, wd_ref, out_ref):
    win = lax.rsqrt(jnp.maximum(degin_ref[:, 0:1], 1.0))
    rows = lax.broadcasted_iota(jnp.int32, (NP, 1), 0)
    maskf = (rows < N).astype(jnp.float32)
    w = w_ref[...]
    bvec = b_ref[...]
    h1 = jnp.maximum(
        jnp.dot(agg_ref[0] * win, w, preferred_element_type=jnp.float32)
        + bvec, 0.0) * maskf
    h2 = jnp.maximum(
        jnp.dot(agg_ref[1] * win, w, preferred_element_type=jnp.float32)
        + bvec, 0.0)
    sv = jax.nn.sigmoid(jnp.sum(h1, axis=0, keepdims=True) / N)
    v = jnp.sum(wd_ref[...] * sv, axis=1, keepdims=True)
    pos = jnp.dot(h1, v, preferred_element_type=jnp.float32)
    neg = jnp.dot(h2, v, preferred_element_type=jnp.float32)

    def softplus(t):
        return jnp.maximum(t, 0.0) + jnp.log1p(jnp.exp(-jnp.abs(t)))

    total = jnp.sum((softplus(-pos) + softplus(neg)) * maskf)
    out_ref[0, 0] = total / (2.0 * N)


def _tc_loss(agg, deg_in, W, b2, Wd):
    return pl.pallas_call(
        _tc_loss_body,
        in_specs=[
            pl.BlockSpec(memory_space=pltpu.VMEM),
            pl.BlockSpec(memory_space=pltpu.VMEM),
            pl.BlockSpec(memory_space=pltpu.VMEM),
            pl.BlockSpec(memory_space=pltpu.VMEM),
            pl.BlockSpec(memory_space=pltpu.VMEM),
        ],
        out_specs=pl.BlockSpec(memory_space=pltpu.SMEM),
        out_shape=jax.ShapeDtypeStruct((1, 1), jnp.float32),
    )(agg, deg_in, W, b2, Wd)


def kernel(x, edge_index, W, b, Wd):
    perm = jax.random.permutation(jax.random.key(42), N).astype(jnp.int32)
    perm_pad = jnp.concatenate([perm, jnp.zeros((NP - N,), jnp.int32)])
    src = edge_index[0]
    dst = edge_index[1]
    srcs = jnp.stack([src, src + NP])
    x_pad = jnp.concatenate([x, jnp.zeros((NP - N, D), x.dtype)])
    zeros128 = jnp.zeros((CH, D), jnp.float32)
    ones128 = jnp.ones((CH, D), jnp.float32)

    deg2d, z = _sc_prep(edge_index.reshape(2 * E), x, perm_pad, zeros128,
                        ones128)
    t = _tc_prep(x_pad, z, deg2d[0])
    agg = _sc_prop(t.reshape(2 * NP, D), srcs.reshape(2 * E), dst, zeros128)
    loss = _tc_loss(agg, deg2d[1], W, b.reshape(1, D), Wd)
    return loss[0, 0]


# speedup vs baseline: 5.2995x; 1.2562x over previous
import jax
import jax.numpy as jnp
from jax import lax
from jax.experimental import pallas as pl
from jax.experimental.pallas import tpu as pltpu
from jax.experimental.pallas import tpu_sc as plsc

N = 10000
E = 320000
D = 128
NC = 2
NS = 16
CH = 128
NP = 10240
EPT = E // NS
EPT2 = 20480
NCHUNK = EPT2 // CH
E2 = EPT2 * NS
RPT = NP // NS
ZCH = 80
ZPT = NP // (NC * NS)
PAD_SRC = 10100
PAD_DST = 10200


_NB = 4


def _sc_prep_body(dedge_hbm, x_hbm, perm_hbm, zeros_hbm, ones_hbm,
                  deg_hbm, z_hbm,
                  deg_acc, idx_all, buf, idxz_v, sem):
    c = lax.axis_index("c")
    s = lax.axis_index("s")
    wid = s * NC + c
    pltpu.sync_copy(dedge_hbm.at[c * NS + s], idx_all)
    pltpu.sync_copy(zeros_hbm, buf)
    for k in range(RPT // CH):
        pltpu.sync_copy(buf, deg_acc.at[pl.ds(s * RPT + k * CH, CH)])
    pltpu.sync_copy(ones_hbm, buf)
    plsc.subcore_barrier()

    def deg_group(g, carry):
        for b in range(_NB):
            ch = g * _NB + b
            pltpu.sync_copy(buf, deg_acc.at[idx_all.at[ch]], add=True)
        return carry

    lax.fori_loop(0, NCHUNK // _NB, deg_group, 0)

    zrows = buf.at[pl.ds(0, ZCH)]

    def z_step(k, carry):
        zb = wid * ZPT + k * ZCH
        pltpu.sync_copy(perm_hbm.at[pl.ds(zb, ZCH)], idxz_v)
        pltpu.async_copy(x_hbm.at[idxz_v], zrows, sem).wait()
        pltpu.sync_copy(zrows, z_hbm.at[pl.ds(zb, ZCH)])
        return carry

    lax.fori_loop(0, ZPT // ZCH, z_step, 0)
    plsc.subcore_barrier()

    for k in range(RPT // CH):
        rb = s * RPT + k * CH
        pltpu.sync_copy(deg_acc.at[pl.ds(rb, CH)], buf)
        pltpu.sync_copy(buf, deg_hbm.at[c, pl.ds(rb, CH)])


_sc_prep = pl.kernel(
    _sc_prep_body,
    out_type=[jax.ShapeDtypeStruct((NC, NP, D), jnp.float32),
              jax.ShapeDtypeStruct((NP, D), jnp.float32)],
    mesh=plsc.VectorSubcoreMesh(core_axis_name="c", subcore_axis_name="s"),
    scratch_types=[
        pltpu.VMEM_SHARED((NP, D), jnp.float32),
        pltpu.VMEM((NCHUNK, CH), jnp.int32),
        pltpu.VMEM((CH, D), jnp.float32),
        pltpu.VMEM((ZCH,), jnp.int32),
        pltpu.SemaphoreType.DMA,
    ],
)


_BCH = 16
_NBLK = NCHUNK // _BCH


def _sc_prop_body(t_hbm, pidx_hbm, zeros_hbm,
                  agg_hbm,
                  acc, iblk, rows, gsem, isem):
    c = lax.axis_index("c")
    s = lax.axis_index("s")
    cs = c * NS + s
    pltpu.sync_copy(zeros_hbm, rows[0])
    for k in range(RPT // CH):
        pltpu.sync_copy(rows[0], acc.at[pl.ds(s * RPT + k * CH, CH)])
    plsc.subcore_barrier()

    pltpu.sync_copy(pidx_hbm.at[cs, pl.ds(0, _BCH)], iblk[0])
    pltpu.async_copy(t_hbm.at[iblk[0].at[0, 0]], rows[0], gsem[0])
    pltpu.async_copy(t_hbm.at[iblk[0].at[1, 0]], rows[1], gsem[1])

    def blockpair(g, carry):
        for bj in range(2):
            m = 2 * g + bj
            cur = iblk[bj]
            oth = iblk[1 - bj]
            last = (bj == 1)

            if not last:
                pltpu.async_copy(
                    pidx_hbm.at[cs, pl.ds((m + 1) * _BCH, _BCH)], oth,
                    isem[1 - bj])
            else:
                @pl.when(g < _NBLK // 2 - 1)
                def _():
                    pltpu.async_copy(
                        pidx_hbm.at[cs, pl.ds((m + 1) * _BCH, _BCH)], oth,
                        isem[1 - bj])

            for j in range(_BCH):
                b = j % 2
                pltpu.make_async_copy(t_hbm.at[cur.at[j, 0]], rows[b],
                                      gsem[b]).wait()
                pltpu.sync_copy(rows[b], acc.at[cur.at[j, 1]], add=True)
                if j < _BCH - 2:
                    pltpu.async_copy(t_hbm.at[cur.at[j + 2, 0]], rows[b],
                                     gsem[b])
                else:
                    if j == _BCH - 2:
                        if not last:
                            pltpu.make_async_copy(
                                pidx_hbm.at[cs, pl.ds((m + 1) * _BCH,
                                                      _BCH)],
                                oth, isem[1 - bj]).wait()
                        else:
                            @pl.when(g < _NBLK // 2 - 1)
                            def _():
                                pltpu.make_async_copy(
                                    pidx_hbm.at[cs, pl.ds((m + 1) * _BCH,
                                                          _BCH)],
                                    oth, isem[1 - bj]).wait()
                    if not last:
                        pltpu.async_copy(
                            t_hbm.at[oth.at[j + 2 - _BCH, 0]], rows[b],
                            gsem[b])
                    else:
                        @pl.when(g < _NBLK // 2 - 1)
                        def _():
                            pltpu.async_copy(
                                t_hbm.at[oth.at[j + 2 - _BCH, 0]], rows[b],
                                gsem[b])
        return carry

    lax.fori_loop(0, _NBLK // 2, blockpair, 0)
    plsc.subcore_barrier()

    for k in range(RPT // CH):
        rb = s * RPT + k * CH
        pltpu.sync_copy(acc.at[pl.ds(rb, CH)], rows[0])
        pltpu.sync_copy(rows[0], agg_hbm.at[c, pl.ds(rb, CH)])


_sc_prop = pl.kernel(
    _sc_prop_body,
    out_type=jax.ShapeDtypeStruct((NC, NP, D), jnp.float32),
    mesh=plsc.VectorSubcoreMesh(core_axis_name="c", subcore_axis_name="s"),
    scratch_types=[
        pltpu.VMEM_SHARED((NP, D), jnp.float32),
        [pltpu.VMEM((_BCH, 2, CH), jnp.int32)] * 2,
        [pltpu.VMEM((CH, D), jnp.float32)] * 2,
        [pltpu.SemaphoreType.DMA] * 2,
        [pltpu.SemaphoreType.DMA] * 2,
    ],
)


_BS = 1024


def _tc_prep_body(x_ref, z_ref, deg_ref, t_ref):
    g = pl.program_id(0)
    u = lax.rsqrt(jnp.maximum(deg_ref[:, 0:1], 1.0))
    rows = lax.broadcasted_iota(jnp.int32, (_BS, 1), 0) + g * _BS
    um = jnp.where(rows < N, u, 0.0)
    t_ref[0] = x_ref[...] * um
    t_ref[1] = z_ref[...] * um


def _tc_prep(x_pad, z, deg_out):
    return pl.pallas_call(
        _tc_prep_body,
        grid=(NP // _BS,),
        in_specs=[
            pl.BlockSpec((_BS, D), lambda g: (g, 0)),
            pl.BlockSpec((_BS, D), lambda g: (g, 0)),
            pl.BlockSpec((_BS, D), lambda g: (g, 0)),
        ],
        out_specs=pl.BlockSpec((2, _BS, D), lambda g: (0, g, 0)),
        out_shape=jax.ShapeDtypeStruct((2, NP, D), jnp.float32),
    )(x_pad, z, deg_out)


def _tc_loss_body(agg_ref, degin_ref, w_ref, b_ref, wd_ref, out_ref):
    win = lax.rsqrt(jnp.maximum(degin_ref[:, 0:1], 1.0))
    rows = lax.broadcasted_iota(jnp.int32, (NP, 1), 0)
    maskf = (rows < N).astype(jnp.float32)
    w = w_ref[...]
    bvec = b_ref[...]
    h1 = jnp.maximum(
        jnp.dot(agg_ref[0] * win, w, preferred_element_type=jnp.float32)
        + bvec, 0.0) * maskf
    h2 = jnp.maximum(
        jnp.dot(agg_ref[1] * win, w, preferred_element_type=jnp.float32)
        + bvec, 0.0)
    sv = jax.nn.sigmoid(jnp.sum(h1, axis=0, keepdims=True) / N)
    v = jnp.sum(wd_ref[...] * sv, axis=1, keepdims=True)
    pos = jnp.dot(h1, v, preferred_element_type=jnp.float32)
    neg = jnp.dot(h2, v, preferred_element_type=jnp.float32)

    def softplus(t):
        return jnp.maximum(t, 0.0) + jnp.log1p(jnp.exp(-jnp.abs(t)))

    total = jnp.sum((softplus(-pos) + softplus(neg)) * maskf)
    out_ref[0, 0] = total / (2.0 * N)


def _tc_loss(agg, deg_in, W, b2, Wd):
    return pl.pallas_call(
        _tc_loss_body,
        in_specs=[
            pl.BlockSpec(memory_space=pltpu.VMEM),
            pl.BlockSpec(memory_space=pltpu.VMEM),
            pl.BlockSpec(memory_space=pltpu.VMEM),
            pl.BlockSpec(memory_space=pltpu.VMEM),
            pl.BlockSpec(memory_space=pltpu.VMEM),
        ],
        out_specs=pl.BlockSpec(memory_space=pltpu.SMEM),
        out_shape=jax.ShapeDtypeStruct((1, 1), jnp.float32),
    )(agg, deg_in, W, b2, Wd)


def kernel(x, edge_index, W, b, Wd):
    perm = jax.random.permutation(jax.random.key(42), N).astype(jnp.int32)
    perm_pad = jnp.concatenate([perm, jnp.zeros((NP - N,), jnp.int32)])
    x_pad = jnp.concatenate([x, jnp.zeros((NP - N, D), x.dtype)])

    src2d = edge_index[0].reshape(NS, EPT)
    dst2d = edge_index[1].reshape(NS, EPT)
    spad = jnp.full((NS, EPT2 - EPT), PAD_SRC, jnp.int32)
    dpad = jnp.full((NS, EPT2 - EPT), PAD_DST, jnp.int32)
    src_t = jnp.concatenate([src2d, spad], axis=1)
    dst_t = jnp.concatenate([dst2d, dpad], axis=1)
    dedge = jnp.concatenate(
        [src_t.reshape(NS, NCHUNK, CH), dst_t.reshape(NS, NCHUNK, CH)])
    sc = src_t.reshape(1, NS, NCHUNK, CH) + jnp.array(
        [0, NP], jnp.int32).reshape(NC, 1, 1, 1)
    dc = jnp.broadcast_to(dst_t.reshape(1, NS, NCHUNK, CH),
                          (NC, NS, NCHUNK, CH))
    pidx = jnp.stack([sc, dc], axis=3).reshape(NC * NS, NCHUNK, 2, CH)

    zeros128 = jnp.zeros((CH, D), jnp.float32)
    ones128 = jnp.ones((CH, D), jnp.float32)

    deg2d, z = _sc_prep(dedge, x, perm_pad, zeros128, ones128)
    t = _tc_prep(x_pad, z, deg2d[0])
    agg = _sc_prop(t.reshape(2 * NP, D), pidx, zeros128)
    loss = _tc_loss(agg, deg2d[1], W, b.reshape(1, D), Wd)
    return loss[0, 0]


# speedup vs baseline: 5.4560x; 1.0295x over previous
import jax
import jax.numpy as jnp
from jax import lax
from jax.experimental import pallas as pl
from jax.experimental.pallas import tpu as pltpu
from jax.experimental.pallas import tpu_sc as plsc

N = 10000
E = 320000
D = 128
NC = 2
NS = 16
CH = 128
NP = 10240
EPT = E // NS
EPT2 = 20480
NCHUNK = EPT2 // CH
PCH = 64
PNCHUNK = EPT2 // PCH
RPT = NP // NS
ZCH = 80
ZPT = NP // (NC * NS)
PAD_SRC = 10100
PAD_DST = 10200


_NB = 4


def _sc_prep_body(dedge_hbm, x_hbm, perm_hbm, zeros_hbm, ones_hbm,
                  deg_hbm, z_hbm,
                  deg_acc, idx_all, buf, idxz_v, sem):
    c = lax.axis_index("c")
    s = lax.axis_index("s")
    wid = s * NC + c
    pltpu.sync_copy(dedge_hbm.at[c * NS + s], idx_all)
    pltpu.sync_copy(zeros_hbm, buf)
    for k in range(RPT // CH):
        pltpu.sync_copy(buf, deg_acc.at[pl.ds(s * RPT + k * CH, CH)])
    pltpu.sync_copy(ones_hbm, buf)
    plsc.subcore_barrier()

    def deg_group(g, carry):
        for b in range(_NB):
            ch = g * _NB + b
            pltpu.sync_copy(buf, deg_acc.at[idx_all.at[ch]], add=True)
        return carry

    lax.fori_loop(0, NCHUNK // _NB, deg_group, 0)

    zrows = buf.at[pl.ds(0, ZCH)]

    def z_step(k, carry):
        zb = wid * ZPT + k * ZCH
        pltpu.sync_copy(perm_hbm.at[pl.ds(zb, ZCH)], idxz_v)
        pltpu.async_copy(x_hbm.at[idxz_v], zrows, sem).wait()
        pltpu.sync_copy(zrows, z_hbm.at[pl.ds(zb, ZCH)])
        return carry

    lax.fori_loop(0, ZPT // ZCH, z_step, 0)
    plsc.subcore_barrier()

    for k in range(RPT // CH):
        rb = s * RPT + k * CH
        pltpu.sync_copy(deg_acc.at[pl.ds(rb, CH)], buf)
        pltpu.sync_copy(buf, deg_hbm.at[c, pl.ds(rb, CH)])


_sc_prep = pl.kernel(
    _sc_prep_body,
    out_type=[jax.ShapeDtypeStruct((NC, NP, D), jnp.float32),
              jax.ShapeDtypeStruct((NP, D), jnp.float32)],
    mesh=plsc.VectorSubcoreMesh(core_axis_name="c", subcore_axis_name="s"),
    scratch_types=[
        pltpu.VMEM_SHARED((NP, D), jnp.float32),
        pltpu.VMEM((NCHUNK, CH), jnp.int32),
        pltpu.VMEM((CH, D), jnp.float32),
        pltpu.VMEM((ZCH,), jnp.int32),
        pltpu.SemaphoreType.DMA,
    ],
)


_BCH = 32
_NBLK = PNCHUNK // _BCH
_LOOK = 4


def _sc_prop_body(t_hbm, pidx_hbm, zeros_hbm,
                  agg_hbm,
                  acc, iblk, rows, gsem, isem):
    c = lax.axis_index("c")
    s = lax.axis_index("s")
    cs = c * NS + s
    pltpu.sync_copy(zeros_hbm.at[pl.ds(0, PCH)], rows[0])
    for k in range(RPT // PCH):
        pltpu.sync_copy(rows[0], acc.at[pl.ds(s * RPT + k * PCH, PCH)])
    plsc.subcore_barrier()

    pltpu.sync_copy(pidx_hbm.at[cs, pl.ds(0, _BCH)], iblk[0])
    for b in range(_LOOK):
        pltpu.async_copy(t_hbm.at[iblk[0].at[b, 0]], rows[b], gsem[b])

    def blockpair(g, carry):
        for bj in range(2):
            m = 2 * g + bj
            cur = iblk[bj]
            oth = iblk[1 - bj]
            last = (bj == 1)
            lastguard = _NBLK // 2 - 1

            if not last:
                pltpu.async_copy(
                    pidx_hbm.at[cs, pl.ds((m + 1) * _BCH, _BCH)], oth,
                    isem[1 - bj])
            else:
                @pl.when(g < lastguard)
                def _():
                    pltpu.async_copy(
                        pidx_hbm.at[cs, pl.ds((m + 1) * _BCH, _BCH)], oth,
                        isem[1 - bj])

            for j in range(_BCH):
                b = j % _LOOK
                pltpu.make_async_copy(t_hbm.at[cur.at[j, 0]], rows[b],
                                      gsem[b]).wait()
                pltpu.sync_copy(rows[b], acc.at[cur.at[j, 1]], add=True)
                if j < _BCH - _LOOK:
                    pltpu.async_copy(t_hbm.at[cur.at[j + _LOOK, 0]],
                                     rows[b], gsem[b])
                else:
                    if j == _BCH - _LOOK:
                        if not last:
                            pltpu.make_async_copy(
                                pidx_hbm.at[cs, pl.ds((m + 1) * _BCH,
                                                      _BCH)],
                                oth, isem[1 - bj]).wait()
                        else:
                            @pl.when(g < lastguard)
                            def _():
                                pltpu.make_async_copy(
                                    pidx_hbm.at[cs, pl.ds((m + 1) * _BCH,
                                                          _BCH)],
                                    oth, isem[1 - bj]).wait()
                    if not last:
                        pltpu.async_copy(
                            t_hbm.at[oth.at[j + _LOOK - _BCH, 0]], rows[b],
                            gsem[b])
                    else:
                        @pl.when(g < lastguard)
                        def _():
                            pltpu.async_copy(
                                t_hbm.at[oth.at[j + _LOOK - _BCH, 0]],
                                rows[b], gsem[b])
        return carry

    lax.fori_loop(0, _NBLK // 2, blockpair, 0)
    plsc.subcore_barrier()

    for k in range(RPT // PCH):
        rb = s * RPT + k * PCH
        pltpu.sync_copy(acc.at[pl.ds(rb, PCH)], rows[0])
        pltpu.sync_copy(rows[0], agg_hbm.at[c, pl.ds(rb, PCH)])


_sc_prop = pl.kernel(
    _sc_prop_body,
    out_type=jax.ShapeDtypeStruct((NC, NP, D), jnp.float32),
    mesh=plsc.VectorSubcoreMesh(core_axis_name="c", subcore_axis_name="s"),
    scratch_types=[
        pltpu.VMEM_SHARED((NP, D), jnp.float32),
        [pltpu.VMEM((_BCH, 2, PCH), jnp.int32)] * 2,
        [pltpu.VMEM((PCH, D), jnp.float32)] * _LOOK,
        [pltpu.SemaphoreType.DMA] * _LOOK,
        [pltpu.SemaphoreType.DMA] * 2,
    ],
)


_BS = 1024


def _tc_prep_body(x_ref, z_ref, deg_ref, t_ref):
    g = pl.program_id(0)
    u = lax.rsqrt(jnp.maximum(deg_ref[:, 0:1], 1.0))
    rows = lax.broadcasted_iota(jnp.int32, (_BS, 1), 0) + g * _BS
    um = jnp.where(rows < N, u, 0.0)
    t_ref[0] = x_ref[...] * um
    t_ref[1] = z_ref[...] * um


def _tc_prep(x_pad, z, deg_out):
    return pl.pallas_call(
        _tc_prep_body,
        grid=(NP // _BS,),
        in_specs=[
            pl.BlockSpec((_BS, D), lambda g: (g, 0)),
            pl.BlockSpec((_BS, D), lambda g: (g, 0)),
            pl.BlockSpec((_BS, D), lambda g: (g, 0)),
        ],
        out_specs=pl.BlockSpec((2, _BS, D), lambda g: (0, g, 0)),
        out_shape=jax.ShapeDtypeStruct((2, NP, D), jnp.float32),
    )(x_pad, z, deg_out)


def _tc_loss_body(agg_ref, degin_ref, w_ref, b_ref, wd_ref, out_ref):
    win = lax.rsqrt(jnp.maximum(degin_ref[:, 0:1], 1.0))
    rows = lax.broadcasted_iota(jnp.int32, (NP, 1), 0)
    maskf = (rows < N).astype(jnp.float32)
    w = w_ref[...]
    bvec = b_ref[...]
    h1 = jnp.maximum(
        jnp.dot(agg_ref[0] * win, w, preferred_element_type=jnp.float32)
        + bvec, 0.0) * maskf
    h2 = jnp.maximum(
        jnp.dot(agg_ref[1] * win, w, preferred_element_type=jnp.float32)
        + bvec, 0.0)
    sv = jax.nn.sigmoid(jnp.sum(h1, axis=0, keepdims=True) / N)
    v = jnp.sum(wd_ref[...] * sv, axis=1, keepdims=True)
    pos = jnp.dot(h1, v, preferred_element_type=jnp.float32)
    neg = jnp.dot(h2, v, preferred_element_type=jnp.float32)

    def softplus(t):
        return jnp.maximum(t, 0.0) + jnp.log1p(jnp.exp(-jnp.abs(t)))

    total = jnp.sum((softplus(-pos) + softplus(neg)) * maskf)
    out_ref[0, 0] = total / (2.0 * N)


def _tc_loss(agg, deg_in, W, b2, Wd):
    return pl.pallas_call(
        _tc_loss_body,
        in_specs=[
            pl.BlockSpec(memory_space=pltpu.VMEM),
            pl.BlockSpec(memory_space=pltpu.VMEM),
            pl.BlockSpec(memory_space=pltpu.VMEM),
            pl.BlockSpec(memory_space=pltpu.VMEM),
            pl.BlockSpec(memory_space=pltpu.VMEM),
        ],
        out_specs=pl.BlockSpec(memory_space=pltpu.SMEM),
        out_shape=jax.ShapeDtypeStruct((1, 1), jnp.float32),
    )(agg, deg_in, W, b2, Wd)


def kernel(x, edge_index, W, b, Wd):
    perm = jax.random.permutation(jax.random.key(42), N).astype(jnp.int32)
    perm_pad = jnp.concatenate([perm, jnp.zeros((NP - N,), jnp.int32)])
    x_pad = jnp.concatenate([x, jnp.zeros((NP - N, D), x.dtype)])

    src2d = edge_index[0].reshape(NS, EPT)
    dst2d = edge_index[1].reshape(NS, EPT)
    spad = jnp.full((NS, EPT2 - EPT), PAD_SRC, jnp.int32)
    dpad = jnp.full((NS, EPT2 - EPT), PAD_DST, jnp.int32)
    src_t = jnp.concatenate([src2d, spad], axis=1)
    dst_t = jnp.concatenate([dst2d, dpad], axis=1)
    dedge = jnp.concatenate(
        [src_t.reshape(NS, NCHUNK, CH), dst_t.reshape(NS, NCHUNK, CH)])
    sc = src_t.reshape(1, NS, PNCHUNK, PCH) + jnp.array(
        [0, NP], jnp.int32).reshape(NC, 1, 1, 1)
    dc = jnp.broadcast_to(dst_t.reshape(1, NS, PNCHUNK, PCH),
                          (NC, NS, PNCHUNK, PCH))
    pidx = jnp.stack([sc, dc], axis=3).reshape(NC * NS, PNCHUNK, 2, PCH)

    zeros128 = jnp.zeros((CH, D), jnp.float32)
    ones128 = jnp.ones((CH, D), jnp.float32)

    deg2d, z = _sc_prep(dedge, x, perm_pad, zeros128, ones128)
    t = _tc_prep(x_pad, z, deg2d[0])
    agg = _sc_prop(t.reshape(2 * NP, D), pidx, zeros128)
    loss = _tc_loss(agg, deg2d[1], W, b.reshape(1, D), Wd)
    return loss[0, 0]
